# baseline (speedup 1.0000x reference)
import functools

import jax
import jax.numpy as jnp
from jax import lax
from jax.experimental import pallas as pl
from jax.experimental.pallas import tpu as pltpu
from jax.experimental.pallas import tpu_sc as plsc

VOCAB = 50358
HIDDEN = 768
BATCH = 4
SEQ = 4096
EPS = 1e-12

NC = 2
NS = 16
NW = NC * NS
ROWS = BATCH * SEQ

NSLAB = 4
CSEQ = SEQ // NSLAB
CROWS = BATCH * CSEQ
BPW = CROWS // NW
CHUNK = 64
NCHUNK = BPW // CHUNK


def _sc_gather(table, ids2, slab):
    mesh = plsc.VectorSubcoreMesh(core_axis_name="c", subcore_axis_name="s")
    nper = CSEQ // BPW

    @functools.partial(
        pl.kernel,
        mesh=mesh,
        out_type=jax.ShapeDtypeStruct((CROWS, HIDDEN), jnp.float32),
        scratch_types=[
            pltpu.VMEM((NCHUNK, CHUNK), jnp.int32),
            pltpu.VMEM((CHUNK, HIDDEN), jnp.float32),
            pltpu.VMEM((CHUNK, HIDDEN), jnp.float32),
            pltpu.SemaphoreType.DMA,
            pltpu.SemaphoreType.DMA,
        ],
    )
    def k(table_hbm, idx_hbm, out_hbm, idx_v, buf0, buf1, sem0, sem1):
        wid = lax.axis_index("s") * NC + lax.axis_index("c")
        base = wid * BPW
        j = ((wid // nper) * (SEQ // CHUNK)
             + slab * (CSEQ // CHUNK)
             + (wid % nper) * NCHUNK)
        pltpu.sync_copy(idx_hbm.at[pl.ds(j, NCHUNK)], idx_v)
        bufs = (buf0, buf1)
        sems = (sem0, sem1)
        handles = [None, None]
        handles[0] = pltpu.async_copy(table_hbm.at[idx_v.at[0]], buf0, sem0)
        for i in range(NCHUNK):
            cur = i % 2
            nxt = 1 - cur
            if i + 1 < NCHUNK:
                handles[nxt] = pltpu.async_copy(
                    table_hbm.at[idx_v.at[i + 1]], bufs[nxt], sems[nxt]
                )
            handles[cur].wait()
            pltpu.sync_copy(bufs[cur], out_hbm.at[pl.ds(base + i * CHUNK, CHUNK)])

    return k(table, ids2)


RB = 1024
NSB = SEQ // RB
SLAB_NSB = CSEQ // RB


def _tc_body(g_ref, pos_ref, ttf_ref, tt_ref, gamma_ref, beta_ref, o_ref):
    x = g_ref[...] + pos_ref[...]
    tt0 = tt_ref[0:1, :]
    tt1 = tt_ref[1:2, :]
    x = x + tt0 + ttf_ref[...].astype(jnp.float32) * (tt1 - tt0)
    mean = jnp.mean(x, axis=1, keepdims=True)
    xc = x - mean
    var = jnp.mean(xc * xc, axis=1, keepdims=True)
    normed = xc * lax.rsqrt(var + EPS)
    o_ref[...] = normed * gamma_ref[...] + beta_ref[...]


def _tc_body_alias(g_ref, pos_ref, ttf_ref, tt_ref, gamma_ref, beta_ref,
                   prev_ref, o_ref):
    del prev_ref
    _tc_body(g_ref, pos_ref, ttf_ref, tt_ref, gamma_ref, beta_ref, o_ref)


def _tc_slab(c, gathered_c, pos, ttf, tt_table, gamma, beta, prev):
    in_specs = [
        pl.BlockSpec((RB, HIDDEN), lambda a, b: (b * SLAB_NSB + a, 0)),
        pl.BlockSpec((RB, HIDDEN), lambda a, b: (c * SLAB_NSB + a, 0)),
        pl.BlockSpec((RB, 1), lambda a, b: (b * NSB + c * SLAB_NSB + a, 0)),
        pl.BlockSpec((2, HIDDEN), lambda a, b: (0, 0)),
        pl.BlockSpec((1, HIDDEN), lambda a, b: (0, 0)),
        pl.BlockSpec((1, HIDDEN), lambda a, b: (0, 0)),
    ]
    args = [gathered_c, pos, ttf, tt_table, gamma, beta]
    aliases = {}
    body = _tc_body
    if prev is not None:
        in_specs.append(pl.BlockSpec(memory_space=pltpu.MemorySpace.HBM))
        args.append(prev)
        aliases = {6: 0}
        body = _tc_body_alias
    return pl.pallas_call(
        body,
        grid=(SLAB_NSB, BATCH),
        in_specs=in_specs,
        out_specs=pl.BlockSpec(
            (RB, HIDDEN), lambda a, b: (b * NSB + c * SLAB_NSB + a, 0)
        ),
        out_shape=jax.ShapeDtypeStruct((ROWS, HIDDEN), jnp.float32),
        input_output_aliases=aliases,
        compiler_params=pltpu.CompilerParams(
            dimension_semantics=("arbitrary", "arbitrary"),
        ),
    )(*args)


def kernel(input_ids, token_type_ids, weight, token_type_embeddings,
           position_embeddings, ln_gamma, ln_beta):
    ids = input_ids.astype(jnp.int32)
    ttf = token_type_ids.astype(jnp.int8).reshape(ROWS, 1)
    gamma = jnp.reshape(ln_gamma, (1, HIDDEN))
    beta = jnp.reshape(ln_beta, (1, HIDDEN))

    ids2 = ids.reshape(ROWS // CHUNK, CHUNK)
    gathered = [_sc_gather(weight, ids2, c) for c in range(NSLAB)]
    out = None
    for c in range(NSLAB):
        out = _tc_slab(c, gathered[c], position_embeddings, ttf,
                       token_type_embeddings, gamma, beta, out)
    return out.reshape(BATCH, SEQ, HIDDEN)


# speedup vs baseline: 1.0102x; 1.0102x over previous
import functools

import jax
import jax.numpy as jnp
from jax import lax
from jax.experimental import pallas as pl
from jax.experimental.pallas import tpu as pltpu
from jax.experimental.pallas import tpu_sc as plsc

VOCAB = 50358
HIDDEN = 768
BATCH = 4
SEQ = 4096
EPS = 1e-12

NC = 2
NS = 16
NW = NC * NS
ROWS = BATCH * SEQ

NSLAB = 2
CSEQ = SEQ // NSLAB
CROWS = BATCH * CSEQ
BPW = CROWS // NW
CHUNK = 64
NCHUNK = BPW // CHUNK


def _sc_gather(table, ids, slab):
    mesh = plsc.VectorSubcoreMesh(core_axis_name="c", subcore_axis_name="s")
    nper = CSEQ // BPW

    @functools.partial(
        pl.kernel,
        mesh=mesh,
        out_type=jax.ShapeDtypeStruct((CROWS, HIDDEN), jnp.float32),
        scratch_types=[
            pltpu.VMEM((NCHUNK, CHUNK), jnp.int32),
            pltpu.VMEM((CHUNK, HIDDEN), jnp.float32),
            pltpu.VMEM((CHUNK, HIDDEN), jnp.float32),
            pltpu.SemaphoreType.DMA,
            pltpu.SemaphoreType.DMA,
        ],
    )
    def k(table_hbm, idx_hbm, out_hbm, idx_v, buf0, buf1, sem0, sem1):
        wid = lax.axis_index("s") * NC + lax.axis_index("c")
        base = wid * BPW
        b = wid // nper
        s0 = slab * CSEQ + (wid % nper) * BPW
        for i in range(NCHUNK):
            pltpu.sync_copy(
                idx_hbm.at[b, pl.ds(s0 + i * CHUNK, CHUNK)], idx_v.at[i]
            )
        bufs = (buf0, buf1)
        sems = (sem0, sem1)
        handles = [None, None]
        handles[0] = pltpu.async_copy(table_hbm.at[idx_v.at[0]], buf0, sem0)
        for i in range(NCHUNK):
            cur = i % 2
            nxt = 1 - cur
            if i + 1 < NCHUNK:
                handles[nxt] = pltpu.async_copy(
                    table_hbm.at[idx_v.at[i + 1]], bufs[nxt], sems[nxt]
                )
            handles[cur].wait()
            pltpu.sync_copy(bufs[cur], out_hbm.at[pl.ds(base + i * CHUNK, CHUNK)])

    return k(table, ids)


RB = 1024
NSB = SEQ // RB
SLAB_NSB = CSEQ // RB


def _tc_body(g_ref, pos_ref, ttf_ref, tt_ref, gamma_ref, beta_ref, o_ref):
    x = g_ref[...] + pos_ref[...]
    tt0 = tt_ref[0:1, :]
    tt1 = tt_ref[1:2, :]
    x = x + tt0 + ttf_ref[...].astype(jnp.float32) * (tt1 - tt0)
    mean = jnp.mean(x, axis=1, keepdims=True)
    xc = x - mean
    var = jnp.mean(xc * xc, axis=1, keepdims=True)
    normed = xc * lax.rsqrt(var + EPS)
    o_ref[...] = normed * gamma_ref[...] + beta_ref[...]


def _tc_body_alias(g_ref, pos_ref, ttf_ref, tt_ref, gamma_ref, beta_ref,
                   prev_ref, o_ref):
    del prev_ref
    _tc_body(g_ref, pos_ref, ttf_ref, tt_ref, gamma_ref, beta_ref, o_ref)


def _tc_slab(c, gathered_c, pos, ttf, tt_table, gamma, beta, prev):
    in_specs = [
        pl.BlockSpec((RB, HIDDEN), lambda a, b: (b * SLAB_NSB + a, 0)),
        pl.BlockSpec((RB, HIDDEN), lambda a, b: (c * SLAB_NSB + a, 0)),
        pl.BlockSpec((RB, 1), lambda a, b: (b * NSB + c * SLAB_NSB + a, 0)),
        pl.BlockSpec((2, HIDDEN), lambda a, b: (0, 0)),
        pl.BlockSpec((1, HIDDEN), lambda a, b: (0, 0)),
        pl.BlockSpec((1, HIDDEN), lambda a, b: (0, 0)),
    ]
    args = [gathered_c, pos, ttf, tt_table, gamma, beta]
    aliases = {}
    body = _tc_body
    if prev is not None:
        in_specs.append(pl.BlockSpec(memory_space=pltpu.MemorySpace.HBM))
        args.append(prev)
        aliases = {6: 0}
        body = _tc_body_alias
    return pl.pallas_call(
        body,
        grid=(SLAB_NSB, BATCH),
        in_specs=in_specs,
        out_specs=pl.BlockSpec(
            (RB, HIDDEN), lambda a, b: (b * NSB + c * SLAB_NSB + a, 0)
        ),
        out_shape=jax.ShapeDtypeStruct((ROWS, HIDDEN), jnp.float32),
        input_output_aliases=aliases,
        compiler_params=pltpu.CompilerParams(
            dimension_semantics=("arbitrary", "arbitrary"),
        ),
    )(*args)


def kernel(input_ids, token_type_ids, weight, token_type_embeddings,
           position_embeddings, ln_gamma, ln_beta):
    ids = input_ids.astype(jnp.int32)
    ttf = token_type_ids.astype(jnp.int8).reshape(ROWS, 1)
    gamma = jnp.reshape(ln_gamma, (1, HIDDEN))
    beta = jnp.reshape(ln_beta, (1, HIDDEN))

    gathered = [_sc_gather(weight, ids, c) for c in range(NSLAB)]
    out = None
    for c in range(NSLAB):
        out = _tc_slab(c, gathered[c], position_embeddings, ttf,
                       token_type_embeddings, gamma, beta, out)
    return out.reshape(BATCH, SEQ, HIDDEN)


# speedup vs baseline: 1.0273x; 1.0170x over previous
import functools

import jax
import jax.numpy as jnp
from jax import lax
from jax.experimental import pallas as pl
from jax.experimental.pallas import tpu as pltpu
from jax.experimental.pallas import tpu_sc as plsc

VOCAB = 50358
HIDDEN = 768
BATCH = 4
SEQ = 4096
EPS = 1e-12

NC = 2
NS = 16
NW = NC * NS
ROWS = BATCH * SEQ

NSLAB = 2
CSEQ = SEQ // NSLAB
CROWS = BATCH * CSEQ
BPW = CROWS // NW
CHUNK = 64
NCHUNK = BPW // CHUNK


def _sc_gather(table, ids, slab):
    mesh = plsc.VectorSubcoreMesh(core_axis_name="c", subcore_axis_name="s")
    nper = CSEQ // BPW

    @functools.partial(
        pl.kernel,
        mesh=mesh,
        out_type=jax.ShapeDtypeStruct((CROWS, HIDDEN), jnp.float32),
        scratch_types=[
            pltpu.VMEM((BPW,), jnp.int32),
            pltpu.VMEM((CHUNK, HIDDEN), jnp.float32),
            pltpu.VMEM((CHUNK, HIDDEN), jnp.float32),
            pltpu.SemaphoreType.DMA,
            pltpu.SemaphoreType.DMA,
        ],
    )
    def k(table_hbm, idx_hbm, out_hbm, idx_v, buf0, buf1, sem0, sem1):
        wid = lax.axis_index("s") * NC + lax.axis_index("c")
        base = wid * BPW
        b = wid // nper
        s0 = slab * CSEQ + (wid % nper) * BPW
        pltpu.sync_copy(idx_hbm.at[b, pl.ds(s0, BPW)], idx_v)
        bufs = (buf0, buf1)
        sems = (sem0, sem1)
        handles = [None, None]

        def chunk_idx(i):
            return idx_v.at[pl.ds(i * CHUNK, CHUNK)]

        handles[0] = pltpu.async_copy(table_hbm.at[chunk_idx(0)], buf0, sem0)
        for i in range(NCHUNK):
            cur = i % 2
            nxt = 1 - cur
            if i + 1 < NCHUNK:
                handles[nxt] = pltpu.async_copy(
                    table_hbm.at[chunk_idx(i + 1)], bufs[nxt], sems[nxt]
                )
            handles[cur].wait()
            pltpu.sync_copy(bufs[cur], out_hbm.at[pl.ds(base + i * CHUNK, CHUNK)])

    return k(table, ids)


RB = 1024
NSB = SEQ // RB
SLAB_NSB = CSEQ // RB


def _tc_body(g_ref, pos_ref, ttf_ref, tt_ref, gamma_ref, beta_ref, o_ref):
    x = g_ref[...] + pos_ref[...]
    tt0 = tt_ref[0:1, :]
    tt1 = tt_ref[1:2, :]
    x = x + tt0 + ttf_ref[...].astype(jnp.float32) * (tt1 - tt0)
    mean = jnp.mean(x, axis=1, keepdims=True)
    xc = x - mean
    var = jnp.mean(xc * xc, axis=1, keepdims=True)
    normed = xc * lax.rsqrt(var + EPS)
    o_ref[...] = normed * gamma_ref[...] + beta_ref[...]


def _tc_body_alias(g_ref, pos_ref, ttf_ref, tt_ref, gamma_ref, beta_ref,
                   prev_ref, o_ref):
    del prev_ref
    _tc_body(g_ref, pos_ref, ttf_ref, tt_ref, gamma_ref, beta_ref, o_ref)


def _tc_slab(c, gathered_c, pos, ttf, tt_table, gamma, beta, prev):
    in_specs = [
        pl.BlockSpec((RB, HIDDEN), lambda a, b: (b * SLAB_NSB + a, 0)),
        pl.BlockSpec((RB, HIDDEN), lambda a, b: (c * SLAB_NSB + a, 0)),
        pl.BlockSpec((RB, 1), lambda a, b: (b * NSB + c * SLAB_NSB + a, 0)),
        pl.BlockSpec((2, HIDDEN), lambda a, b: (0, 0)),
        pl.BlockSpec((1, HIDDEN), lambda a, b: (0, 0)),
        pl.BlockSpec((1, HIDDEN), lambda a, b: (0, 0)),
    ]
    args = [gathered_c, pos, ttf, tt_table, gamma, beta]
    aliases = {}
    body = _tc_body
    if prev is not None:
        in_specs.append(pl.BlockSpec(memory_space=pltpu.MemorySpace.HBM))
        args.append(prev)
        aliases = {6: 0}
        body = _tc_body_alias
    return pl.pallas_call(
        body,
        grid=(SLAB_NSB, BATCH),
        in_specs=in_specs,
        out_specs=pl.BlockSpec(
            (RB, HIDDEN), lambda a, b: (b * NSB + c * SLAB_NSB + a, 0)
        ),
        out_shape=jax.ShapeDtypeStruct((ROWS, HIDDEN), jnp.float32),
        input_output_aliases=aliases,
        compiler_params=pltpu.CompilerParams(
            dimension_semantics=("arbitrary", "arbitrary"),
        ),
    )(*args)


def kernel(input_ids, token_type_ids, weight, token_type_embeddings,
           position_embeddings, ln_gamma, ln_beta):
    ids = input_ids.astype(jnp.int32)
    ttf = token_type_ids.astype(jnp.int8).reshape(ROWS, 1)
    gamma = jnp.reshape(ln_gamma, (1, HIDDEN))
    beta = jnp.reshape(ln_beta, (1, HIDDEN))

    gathered = [_sc_gather(weight, ids, c) for c in range(NSLAB)]
    out = None
    for c in range(NSLAB):
        out = _tc_slab(c, gathered[c], position_embeddings, ttf,
                       token_type_embeddings, gamma, beta, out)
    return out.reshape(BATCH, SEQ, HIDDEN)
